# Initial kernel scaffold; baseline (speedup 1.0000x reference)
import jax
import jax.numpy as jnp
from jax.experimental import pallas as pl


def kernel(x, feature_mtx_static, hlvs, batch, edge_index, eps, conv_W, conv_b, node_W, node_b, hlv_W, hlv_b):
    raise NotImplementedError("write your pallas kernel here")


# speedup vs baseline: 9.1003x; 9.1003x over previous
import functools

import jax
import jax.numpy as jnp
import numpy as np
from jax import lax
from jax.experimental import pallas as pl
from jax.experimental.pallas import tpu as pltpu
from jax.experimental.pallas import tpu_sc as plsc

NUM_CORES = 2
NUM_SUBCORES = 16
LANES = 16
EDGE_LANES = 128
K_GROUP = 16
ZROWS = 256

F32 = jnp.float32


def _ceil_to(x, m):
    return (x + m - 1) // m * m


def _make_edge_segsum(half, rows_per_tile, feat):
    mesh = plsc.VectorSubcoreMesh(core_axis_name="c", subcore_axis_name="s")
    groups = rows_per_tile // K_GROUP
    n_acc = half + LANES
    zpt = n_acc // NUM_SUBCORES
    wpt = half // NUM_SUBCORES

    @functools.partial(
        pl.kernel,
        out_type=jax.ShapeDtypeStruct((NUM_CORES, half, feat), F32),
        mesh=mesh,
        scratch_types=[
            pltpu.VMEM((K_GROUP, EDGE_LANES), jnp.int32),
            pltpu.VMEM((K_GROUP, EDGE_LANES), jnp.int32),
            pltpu.VMEM((K_GROUP, EDGE_LANES, feat), F32),
            pltpu.VMEM((ZROWS, feat), F32),
            pltpu.VMEM_SHARED((n_acc, feat), F32),
            pltpu.SemaphoreType.DMA,
            pltpu.SemaphoreType.DMA,
        ],
        compiler_params=pltpu.CompilerParams(use_tc_tiling_on_sc=False),
    )
    def seg(a_hbm, src_hbm, dst_hbm, out_hbm, src_v, dst_v, rows_v, zbuf_v,
            acc_sh, sem_g, sem_s):
        c = lax.axis_index("c")
        s = lax.axis_index("s")
        lo = c * half
        junk = jnp.uint32(half + 1)

        def zfill(i, _):
            zbuf_v[i] = jnp.zeros((feat,), F32)
            return 0

        lax.fori_loop(0, ZROWS, zfill, 0)
        zrow0 = s * zpt
        nz, rem = zpt // ZROWS, zpt % ZROWS
        zcopies = [
            pltpu.async_copy(
                zbuf_v, acc_sh.at[pl.ds(zrow0 + k * ZROWS, ZROWS)], sem_s)
            for k in range(nz)
        ]
        if rem:
            zcopies.append(
                pltpu.async_copy(
                    zbuf_v.at[pl.ds(0, rem)],
                    acc_sh.at[pl.ds(zrow0 + nz * ZROWS, rem)], sem_s))
        for d in zcopies:
            d.wait()
        plsc.subcore_barrier()

        row0 = s * rows_per_tile

        def grp(g, _):
            base = row0 + g * K_GROUP
            pltpu.sync_copy(src_hbm.at[pl.ds(base, K_GROUP)], src_v)
            pltpu.sync_copy(dst_hbm.at[pl.ds(base, K_GROUP)], dst_v)
            gs = [
                pltpu.async_copy(a_hbm.at[src_v.at[j]], rows_v.at[j], sem_g)
                for j in range(K_GROUP)
            ]
            for j in range(K_GROUP):
                for k in range(EDGE_LANES // LANES):
                    d = dst_v[j, pl.ds(k * LANES, LANES)]
                    dl = lax.bitcast_convert_type(d - lo, jnp.uint32)
                    dl = lax.bitcast_convert_type(
                        jnp.minimum(dl, junk), jnp.int32)
                    dst_v[j, pl.ds(k * LANES, LANES)] = dl
            for d in gs:
                d.wait()
            ss = [
                pltpu.async_copy(
                    rows_v.at[j], acc_sh.at[dst_v.at[j]], sem_s, add=True)
                for j in range(K_GROUP)
            ]
            for d in ss:
                d.wait()
            return 0

        lax.fori_loop(0, groups, grp, 0)
        plsc.subcore_barrier()

        pltpu.sync_copy(
            acc_sh.at[pl.ds(s * wpt, wpt)],
            out_hbm.at[c, pl.ds(s * wpt, wpt)])

    return seg


def _mlp_refs(h, w_refs, b_refs):
    for w_ref, b_ref in zip(w_refs, b_refs):
        h = jnp.maximum(jnp.dot(h, w_ref[...],
                                preferred_element_type=F32) + b_ref[...], 0.0)
    return h


def _col_proj(cols_from, total, offset):
    r = lax.broadcasted_iota(jnp.int32, (cols_from, total), 0)
    c = lax.broadcasted_iota(jnp.int32, (cols_from, total), 1)
    return (c - offset == r).astype(F32)


def _prep_body(x_ref, st_ref, b_ref, hlvs_ref, out_ref, *, blk, g, n_all):
    onehot = (b_ref[...] == lax.broadcasted_iota(
        jnp.int32, (blk, g), 1)).astype(F32)
    hlv_pn = jnp.dot(onehot, hlvs_ref[...], preferred_element_type=F32)
    n_node = st_ref.shape[1]
    n_hlvs = hlvs_ref.shape[1]
    out_ref[...] = (
        jnp.dot(x_ref[...], _col_proj(1, n_all, 0),
                preferred_element_type=F32)
        + jnp.dot(st_ref[...], _col_proj(n_node, n_all, 4),
                  preferred_element_type=F32)
        + jnp.dot(hlv_pn, _col_proj(n_hlvs, n_all, 4 + n_node),
                  preferred_element_type=F32))


def _round_core(a_ref, p_ref, eps_ref, wrefs):
    (cw1, cw2, cw3, cw4, cb1, cb2, cb3, cb4,
     nw1, nw2, nw3, nw4, nb1, nb2, nb3, nb4) = wrefs
    a = a_ref[...]
    n_all = a.shape[1]
    m = (1.0 + eps_ref[0, 0]) * a + p_ref[...]
    h = _mlp_refs(m, (cw1, cw2, cw3, cw4), (cb1, cb2, cb3, cb4))
    n_dyn = h.shape[1]
    a_static = jnp.dot(a, _static_sel(n_all, n_dyn),
                       preferred_element_type=F32)
    z = jnp.maximum(
        jnp.dot(h, nw1[...][:n_dyn], preferred_element_type=F32)
        + jnp.dot(a_static, nw1[...][n_dyn:], preferred_element_type=F32)
        + nb1[...], 0.0)
    h2 = _mlp_refs(z, (nw2, nw3, nw4), (nb2, nb3, nb4))
    return h2, a_static


def _static_sel(n_all, n_dyn):
    r = lax.broadcasted_iota(jnp.int32, (n_all, n_all - n_dyn), 0)
    c = lax.broadcasted_iota(jnp.int32, (n_all, n_all - n_dyn), 1)
    return (r - n_dyn == c).astype(F32)


def _round1_body(a_ref, p_ref, eps_ref, *wrefs_and_out):
    *wrefs, out_ref = wrefs_and_out
    h2, a_static = _round_core(a_ref, p_ref, eps_ref, wrefs)
    n_all = a_ref.shape[1]
    n_dyn = h2.shape[1]
    out_ref[...] = (
        jnp.dot(h2, _col_proj(n_dyn, n_all, 0), preferred_element_type=F32)
        + jnp.dot(a_static, _col_proj(n_all - n_dyn, n_all, n_dyn),
                  preferred_element_type=F32))


def _round2_body(a_ref, p_ref, eps_ref, bt_ref, hlvs_ref, *rest, nblocks, g):
    (cw1, cw2, cw3, cw4, cb1, cb2, cb3, cb4,
     nw1, nw2, nw3, nw4, nb1, nb2, nb3, nb4,
     hw1, hw2, hw3, hw4, hb1, hb2, hb3, hb4,
     pooled_ref, out_ref) = rest
    wrefs = (cw1, cw2, cw3, cw4, cb1, cb2, cb3, cb4,
             nw1, nw2, nw3, nw4, nb1, nb2, nb3, nb4)
    h2, _ = _round_core(a_ref, p_ref, eps_ref, wrefs)
    blk = h2.shape[0]
    i = pl.program_id(0)

    onehot_t = (bt_ref[0] == lax.broadcasted_iota(
        jnp.int32, (g, blk), 0)).astype(F32)
    partial = jnp.dot(onehot_t, h2, preferred_element_type=F32)

    @pl.when(i == 0)
    def _():
        pooled_ref[...] = jnp.zeros_like(pooled_ref)

    pooled_ref[...] += partial

    @pl.when(i == nblocks - 1)
    def _():
        pooled = pooled_ref[...]
        n_hlvs = hlvs_ref.shape[1]
        n_dyn = pooled.shape[1]
        tot = n_hlvs + n_dyn
        z = (jnp.dot(hlvs_ref[...], _col_proj(n_hlvs, tot, 0),
                     preferred_element_type=F32)
             + jnp.dot(pooled, _col_proj(n_dyn, tot, n_hlvs),
                       preferred_element_type=F32))
        out_ref[...] = _mlp_refs(z, (hw1, hw2, hw3, hw4),
                                 (hb1, hb2, hb3, hb4))


def kernel(x, feature_mtx_static, hlvs, batch, edge_index, eps,
           conv_W, conv_b, node_W, node_b, hlv_W, hlv_b):
    n = x.shape[0]
    e = edge_index.shape[1]
    g = hlvs.shape[0]
    n_node = feature_mtx_static.shape[1]
    n_hlvs = hlvs.shape[1]
    n_dyn = conv_W[-1].shape[1]
    n_all = n_dyn + n_node + n_hlvs

    blk = 2000
    nblocks = n // blk
    assert nblocks * blk == n

    n_rows = -(-e // EDGE_LANES)
    rows_per_tile = _ceil_to(-(-n_rows // NUM_SUBCORES), K_GROUP)
    r_pad = rows_per_tile * NUM_SUBCORES
    pad_e = r_pad * EDGE_LANES - e
    half = n // 2
    assert 2 * half == n and half % NUM_SUBCORES == 0
    assert (half + LANES) % NUM_SUBCORES == 0
    src2d = jnp.concatenate(
        [edge_index[0], jnp.zeros((pad_e,), jnp.int32)]).reshape(
            r_pad, EDGE_LANES)
    dst2d = jnp.concatenate(
        [edge_index[1], jnp.full((pad_e,), n, jnp.int32)]).reshape(
            r_pad, EDGE_LANES)

    segsum = _make_edge_segsum(half, rows_per_tile, n_all)

    full = lambda shape: pl.BlockSpec(shape, lambda i: (0,) * len(shape))
    w_in = []
    w_ops = []
    for W, b in ((conv_W, conv_b), (node_W, node_b)):
        for arr in (*W, *(jnp.reshape(bb, (1, -1)) for bb in b)):
            w_ops.append(arr)
            w_in.append(full(arr.shape))
    eps2d = jnp.reshape(eps, (1, 1))

    a_spec = pl.BlockSpec((blk, n_all), lambda i: (i, 0))
    p_spec = a_spec

    a0 = pl.pallas_call(
        functools.partial(_prep_body, blk=blk, g=g, n_all=n_all),
        grid=(nblocks,),
        in_specs=[
            pl.BlockSpec((blk, 1), lambda i: (i, 0)),
            pl.BlockSpec((blk, n_node), lambda i: (i, 0)),
            pl.BlockSpec((blk, 1), lambda i: (i, 0)),
            full((g, n_hlvs)),
        ],
        out_specs=a_spec,
        out_shape=jax.ShapeDtypeStruct((n, n_all), F32),
    )(x, feature_mtx_static, jnp.reshape(batch, (n, 1)), hlvs)

    p0 = segsum(a0, src2d, dst2d).reshape(n, n_all)
    a1 = pl.pallas_call(
        _round1_body,
        grid=(nblocks,),
        in_specs=[a_spec, p_spec, full((1, 1))] + w_in,
        out_specs=a_spec,
        out_shape=jax.ShapeDtypeStruct((n, n_all), F32),
    )(a0, p0, eps2d, *w_ops)

    p1 = segsum(a1, src2d, dst2d).reshape(n, n_all)
    hw_ops = list(hlv_W) + [jnp.reshape(bb, (1, -1)) for bb in hlv_b]
    hw_in = [full(arr.shape) for arr in hw_ops]
    _, out = pl.pallas_call(
        functools.partial(_round2_body, nblocks=nblocks, g=g),
        grid=(nblocks,),
        in_specs=[a_spec, p_spec, full((1, 1)),
                  pl.BlockSpec((1, 1, blk), lambda i: (i, 0, 0)),
                  full((g, n_hlvs))] + w_in + hw_in,
        out_specs=[full((g, n_dyn)), full((g, 1))],
        out_shape=[jax.ShapeDtypeStruct((g, n_dyn), F32),
                   jax.ShapeDtypeStruct((g, 1), F32)],
    )(a1, p1, eps2d, jnp.reshape(batch, (nblocks, 1, blk)), hlvs,
      *w_ops, *hw_ops)
    return out


# speedup vs baseline: 21.1170x; 2.3205x over previous
import functools

import jax
import jax.numpy as jnp
import numpy as np
from jax import lax
from jax.experimental import pallas as pl
from jax.experimental.pallas import tpu as pltpu
from jax.experimental.pallas import tpu_sc as plsc

NUM_CORES = 2
NUM_SUBCORES = 16
LANES = 16
EDGE_LANES = 128
K_GROUP = 16
ZROWS = 256

F32 = jnp.float32


def _ceil_to(x, m):
    return (x + m - 1) // m * m


def _make_edge_segsum(half, rows_per_tile, feat):
    mesh = plsc.VectorSubcoreMesh(core_axis_name="c", subcore_axis_name="s")
    groups = rows_per_tile // K_GROUP
    n_acc = half + LANES
    zpt = n_acc // NUM_SUBCORES
    wpt = half // NUM_SUBCORES

    @functools.partial(
        pl.kernel,
        out_type=jax.ShapeDtypeStruct((NUM_CORES, half, feat), F32),
        mesh=mesh,
        scratch_types=[
            pltpu.VMEM((K_GROUP, EDGE_LANES), jnp.int32),
            pltpu.VMEM((K_GROUP, EDGE_LANES), jnp.int32),
            pltpu.VMEM((K_GROUP, EDGE_LANES, feat), F32),
            pltpu.VMEM((ZROWS, feat), F32),
            pltpu.VMEM_SHARED((n_acc, feat), F32),
            pltpu.SemaphoreType.DMA,
            pltpu.SemaphoreType.DMA,
        ],
        compiler_params=pltpu.CompilerParams(use_tc_tiling_on_sc=False),
    )
    def seg(a_hbm, src_hbm, dst_hbm, out_hbm, src_v, dst_v, rows_v, zbuf_v,
            acc_sh, sem_g, sem_s):
        c = lax.axis_index("c")
        s = lax.axis_index("s")
        lo = c * half

        def zfill(i, _):
            zbuf_v[i] = jnp.zeros((feat,), F32)
            return 0

        lax.fori_loop(0, ZROWS, zfill, 0)
        zrow0 = s * zpt
        nz, rem = zpt // ZROWS, zpt % ZROWS
        zcopies = [
            pltpu.async_copy(
                zbuf_v, acc_sh.at[pl.ds(zrow0 + k * ZROWS, ZROWS)], sem_s)
            for k in range(nz)
        ]
        if rem:
            zcopies.append(
                pltpu.async_copy(
                    zbuf_v.at[pl.ds(0, rem)],
                    acc_sh.at[pl.ds(zrow0 + nz * ZROWS, rem)], sem_s))
        for d in zcopies:
            d.wait()
        plsc.subcore_barrier()

        row0 = s * rows_per_tile

        def grp(g, _):
            base = row0 + g * K_GROUP
            pltpu.sync_copy(src_hbm.at[pl.ds(base, K_GROUP)], src_v)
            pltpu.sync_copy(dst_hbm.at[pl.ds(base, K_GROUP)], dst_v)
            gs = [
                pltpu.async_copy(a_hbm.at[src_v.at[j]], rows_v.at[j], sem_g)
                for j in range(K_GROUP)
            ]
            for j in range(K_GROUP):
                for k in range(EDGE_LANES // LANES):
                    d = dst_v[j, pl.ds(k * LANES, LANES)]
                    dl = d - lo
                    inb = jnp.less(
                        lax.bitcast_convert_type(dl, jnp.uint32),
                        jnp.uint32(half))
                    dst_v[j, pl.ds(k * LANES, LANES)] = jnp.where(
                        inb, dl, half + (d & (LANES - 1)))
            for d in gs:
                d.wait()
            ss = [
                pltpu.async_copy(
                    rows_v.at[j], acc_sh.at[dst_v.at[j]], sem_s, add=True)
                for j in range(K_GROUP)
            ]
            for d in ss:
                d.wait()
            return 0

        lax.fori_loop(0, groups, grp, 0)
        plsc.subcore_barrier()

        pltpu.sync_copy(
            acc_sh.at[pl.ds(s * wpt, wpt)],
            out_hbm.at[c, pl.ds(s * wpt, wpt)])

    return seg


def _mlp_refs(h, w_refs, b_refs):
    for w_ref, b_ref in zip(w_refs, b_refs):
        h = jnp.maximum(jnp.dot(h, w_ref[...],
                                preferred_element_type=F32) + b_ref[...], 0.0)
    return h


def _col_proj(cols_from, total, offset):
    r = lax.broadcasted_iota(jnp.int32, (cols_from, total), 0)
    c = lax.broadcasted_iota(jnp.int32, (cols_from, total), 1)
    return (c - offset == r).astype(F32)


def _prep_body(x_ref, st_ref, b_ref, hlvs_ref, out_ref, *, blk, g, n_all):
    onehot = (b_ref[...] == lax.broadcasted_iota(
        jnp.int32, (blk, g), 1)).astype(F32)
    hlv_pn = jnp.dot(onehot, hlvs_ref[...], preferred_element_type=F32)
    n_node = st_ref.shape[1]
    n_hlvs = hlvs_ref.shape[1]
    out_ref[...] = (
        jnp.dot(x_ref[...], _col_proj(1, n_all, 0),
                preferred_element_type=F32)
        + jnp.dot(st_ref[...], _col_proj(n_node, n_all, 4),
                  preferred_element_type=F32)
        + jnp.dot(hlv_pn, _col_proj(n_hlvs, n_all, 4 + n_node),
                  preferred_element_type=F32))


def _round_core(a_ref, p_ref, eps_ref, wrefs):
    (cw1, cw2, cw3, cw4, cb1, cb2, cb3, cb4,
     nw1, nw2, nw3, nw4, nb1, nb2, nb3, nb4) = wrefs
    a = a_ref[...]
    n_all = a.shape[1]
    m = (1.0 + eps_ref[0, 0]) * a + p_ref[...]
    h = _mlp_refs(m, (cw1, cw2, cw3, cw4), (cb1, cb2, cb3, cb4))
    n_dyn = h.shape[1]
    a_static = jnp.dot(a, _static_sel(n_all, n_dyn),
                       preferred_element_type=F32)
    z = jnp.maximum(
        jnp.dot(h, nw1[...][:n_dyn], preferred_element_type=F32)
        + jnp.dot(a_static, nw1[...][n_dyn:], preferred_element_type=F32)
        + nb1[...], 0.0)
    h2 = _mlp_refs(z, (nw2, nw3, nw4), (nb2, nb3, nb4))
    return h2, a_static


def _static_sel(n_all, n_dyn):
    r = lax.broadcasted_iota(jnp.int32, (n_all, n_all - n_dyn), 0)
    c = lax.broadcasted_iota(jnp.int32, (n_all, n_all - n_dyn), 1)
    return (r - n_dyn == c).astype(F32)


def _round1_body(a_ref, p_ref, eps_ref, *wrefs_and_out):
    *wrefs, out_ref = wrefs_and_out
    h2, a_static = _round_core(a_ref, p_ref, eps_ref, wrefs)
    n_all = a_ref.shape[1]
    n_dyn = h2.shape[1]
    out_ref[...] = (
        jnp.dot(h2, _col_proj(n_dyn, n_all, 0), preferred_element_type=F32)
        + jnp.dot(a_static, _col_proj(n_all - n_dyn, n_all, n_dyn),
                  preferred_element_type=F32))


def _round2_body(a_ref, p_ref, eps_ref, bt_ref, hlvs_ref, *rest, nblocks, g):
    (cw1, cw2, cw3, cw4, cb1, cb2, cb3, cb4,
     nw1, nw2, nw3, nw4, nb1, nb2, nb3, nb4,
     hw1, hw2, hw3, hw4, hb1, hb2, hb3, hb4,
     pooled_ref, out_ref) = rest
    wrefs = (cw1, cw2, cw3, cw4, cb1, cb2, cb3, cb4,
             nw1, nw2, nw3, nw4, nb1, nb2, nb3, nb4)
    h2, _ = _round_core(a_ref, p_ref, eps_ref, wrefs)
    blk = h2.shape[0]
    i = pl.program_id(0)

    onehot_t = (bt_ref[0] == lax.broadcasted_iota(
        jnp.int32, (g, blk), 0)).astype(F32)
    partial = jnp.dot(onehot_t, h2, preferred_element_type=F32)

    @pl.when(i == 0)
    def _():
        pooled_ref[...] = jnp.zeros_like(pooled_ref)

    pooled_ref[...] += partial

    @pl.when(i == nblocks - 1)
    def _():
        pooled = pooled_ref[...]
        n_hlvs = hlvs_ref.shape[1]
        n_dyn = pooled.shape[1]
        tot = n_hlvs + n_dyn
        z = (jnp.dot(hlvs_ref[...], _col_proj(n_hlvs, tot, 0),
                     preferred_element_type=F32)
             + jnp.dot(pooled, _col_proj(n_dyn, tot, n_hlvs),
                       preferred_element_type=F32))
        out_ref[...] = _mlp_refs(z, (hw1, hw2, hw3, hw4),
                                 (hb1, hb2, hb3, hb4))


def kernel(x, feature_mtx_static, hlvs, batch, edge_index, eps,
           conv_W, conv_b, node_W, node_b, hlv_W, hlv_b):
    n = x.shape[0]
    e = edge_index.shape[1]
    g = hlvs.shape[0]
    n_node = feature_mtx_static.shape[1]
    n_hlvs = hlvs.shape[1]
    n_dyn = conv_W[-1].shape[1]
    n_all = n_dyn + n_node + n_hlvs

    blk = 2000
    nblocks = n // blk
    assert nblocks * blk == n

    n_rows = -(-e // EDGE_LANES)
    rows_per_tile = _ceil_to(-(-n_rows // NUM_SUBCORES), K_GROUP)
    r_pad = rows_per_tile * NUM_SUBCORES
    pad_e = r_pad * EDGE_LANES - e
    half = n // 2
    assert 2 * half == n and half % NUM_SUBCORES == 0
    assert (half + LANES) % NUM_SUBCORES == 0
    src2d = jnp.concatenate(
        [edge_index[0], jnp.zeros((pad_e,), jnp.int32)]).reshape(
            r_pad, EDGE_LANES)
    dst2d = jnp.concatenate(
        [edge_index[1], jnp.full((pad_e,), n, jnp.int32)]).reshape(
            r_pad, EDGE_LANES)

    segsum = _make_edge_segsum(half, rows_per_tile, n_all)

    full = lambda shape: pl.BlockSpec(shape, lambda i: (0,) * len(shape))
    w_in = []
    w_ops = []
    for W, b in ((conv_W, conv_b), (node_W, node_b)):
        for arr in (*W, *(jnp.reshape(bb, (1, -1)) for bb in b)):
            w_ops.append(arr)
            w_in.append(full(arr.shape))
    eps2d = jnp.reshape(eps, (1, 1))

    a_spec = pl.BlockSpec((blk, n_all), lambda i: (i, 0))
    p_spec = a_spec

    a0 = pl.pallas_call(
        functools.partial(_prep_body, blk=blk, g=g, n_all=n_all),
        grid=(nblocks,),
        in_specs=[
            pl.BlockSpec((blk, 1), lambda i: (i, 0)),
            pl.BlockSpec((blk, n_node), lambda i: (i, 0)),
            pl.BlockSpec((blk, 1), lambda i: (i, 0)),
            full((g, n_hlvs)),
        ],
        out_specs=a_spec,
        out_shape=jax.ShapeDtypeStruct((n, n_all), F32),
    )(x, feature_mtx_static, jnp.reshape(batch, (n, 1)), hlvs)

    p0 = segsum(a0, src2d, dst2d).reshape(n, n_all)
    a1 = pl.pallas_call(
        _round1_body,
        grid=(nblocks,),
        in_specs=[a_spec, p_spec, full((1, 1))] + w_in,
        out_specs=a_spec,
        out_shape=jax.ShapeDtypeStruct((n, n_all), F32),
    )(a0, p0, eps2d, *w_ops)

    p1 = segsum(a1, src2d, dst2d).reshape(n, n_all)
    hw_ops = list(hlv_W) + [jnp.reshape(bb, (1, -1)) for bb in hlv_b]
    hw_in = [full(arr.shape) for arr in hw_ops]
    _, out = pl.pallas_call(
        functools.partial(_round2_body, nblocks=nblocks, g=g),
        grid=(nblocks,),
        in_specs=[a_spec, p_spec, full((1, 1)),
                  pl.BlockSpec((1, 1, blk), lambda i: (i, 0, 0)),
                  full((g, n_hlvs))] + w_in + hw_in,
        out_specs=[full((g, n_dyn)), full((g, 1))],
        out_shape=[jax.ShapeDtypeStruct((g, n_dyn), F32),
                   jax.ShapeDtypeStruct((g, 1), F32)],
    )(a1, p1, eps2d, jnp.reshape(batch, (nblocks, 1, blk)), hlvs,
      *w_ops, *hw_ops)
    return out


# speedup vs baseline: 22.8731x; 1.0832x over previous
import functools

import jax
import jax.numpy as jnp
from jax import lax
from jax.experimental import pallas as pl
from jax.experimental.pallas import tpu as pltpu
from jax.experimental.pallas import tpu_sc as plsc

NUM_CORES = 2
NUM_SUBCORES = 16
LANES = 16
EDGE_LANES = 128
K_GROUP = 16
ZROWS = 256

BLK = 2048

F32 = jnp.float32


def _ceil_to(x, m):
    return (x + m - 1) // m * m


def _make_edge_segsum(half, rows_per_tile, feat):
    mesh = plsc.VectorSubcoreMesh(core_axis_name="c", subcore_axis_name="s")
    groups = rows_per_tile // K_GROUP
    n_acc = half + LANES
    zpt = n_acc // NUM_SUBCORES
    wpt = half // NUM_SUBCORES

    @functools.partial(
        pl.kernel,
        out_type=jax.ShapeDtypeStruct((NUM_CORES, half, feat), F32),
        mesh=mesh,
        scratch_types=[
            pltpu.VMEM((K_GROUP, EDGE_LANES), jnp.int32),
            pltpu.VMEM((K_GROUP, EDGE_LANES), jnp.int32),
            pltpu.VMEM((K_GROUP, EDGE_LANES, feat), F32),
            pltpu.VMEM((ZROWS, feat), F32),
            pltpu.VMEM_SHARED((n_acc, feat), F32),
            pltpu.SemaphoreType.DMA,
            pltpu.SemaphoreType.DMA,
        ],
        compiler_params=pltpu.CompilerParams(use_tc_tiling_on_sc=False),
    )
    def seg(a_hbm, src_hbm, dst_hbm, out_hbm, src_v, dst_v, rows_v, zbuf_v,
            acc_sh, sem_g, sem_s):
        c = lax.axis_index("c")
        s = lax.axis_index("s")
        lo = c * half

        def zfill(i, _):
            zbuf_v[i] = jnp.zeros((feat,), F32)
            return 0

        lax.fori_loop(0, ZROWS, zfill, 0)
        zrow0 = s * zpt
        nz, rem = zpt // ZROWS, zpt % ZROWS
        zcopies = [
            pltpu.async_copy(
                zbuf_v, acc_sh.at[pl.ds(zrow0 + k * ZROWS, ZROWS)], sem_s)
            for k in range(nz)
        ]
        if rem:
            zcopies.append(
                pltpu.async_copy(
                    zbuf_v.at[pl.ds(0, rem)],
                    acc_sh.at[pl.ds(zrow0 + nz * ZROWS, rem)], sem_s))
        for d in zcopies:
            d.wait()
        plsc.subcore_barrier()

        row0 = s * rows_per_tile

        def grp(g, _):
            base = row0 + g * K_GROUP
            pltpu.sync_copy(src_hbm.at[pl.ds(base, K_GROUP)], src_v)
            pltpu.sync_copy(dst_hbm.at[pl.ds(base, K_GROUP)], dst_v)
            gs = [
                pltpu.async_copy(a_hbm.at[src_v.at[j]], rows_v.at[j], sem_g)
                for j in range(K_GROUP)
            ]
            for j in range(K_GROUP):
                for k in range(EDGE_LANES // LANES):
                    d = dst_v[j, pl.ds(k * LANES, LANES)]
                    dl = d - lo
                    inb = jnp.less(
                        lax.bitcast_convert_type(dl, jnp.uint32),
                        jnp.uint32(half))
                    dst_v[j, pl.ds(k * LANES, LANES)] = jnp.where(
                        inb, dl, half + (d & (LANES - 1)))
            for d in gs:
                d.wait()
            ss = [
                pltpu.async_copy(
                    rows_v.at[j], acc_sh.at[dst_v.at[j]], sem_s, add=True)
                for j in range(K_GROUP)
            ]
            for d in ss:
                d.wait()
            return 0

        lax.fori_loop(0, groups, grp, 0)
        plsc.subcore_barrier()

        pltpu.sync_copy(
            acc_sh.at[pl.ds(s * wpt, wpt)],
            out_hbm.at[c, pl.ds(s * wpt, wpt)])

    return seg


def _row_proj(total, rows_from, offset):
    r = lax.broadcasted_iota(jnp.int32, (total, rows_from), 0)
    c = lax.broadcasted_iota(jnp.int32, (total, rows_from), 1)
    return (r - offset == c).astype(F32)


def _mlpT(h, w_refs, b_refs):
    for w_ref, b_ref in zip(w_refs, b_refs):
        h = jnp.maximum(jnp.dot(w_ref[...], h,
                                preferred_element_type=F32) + b_ref[...], 0.0)
    return h


def _prep_body(xT_ref, stT_ref, bT_ref, hlvsT_ref, outT_ref, *, g):
    b = xT_ref.shape[1]
    oh = (lax.broadcasted_iota(jnp.int32, (g, b), 0)
          == bT_ref[...]).astype(F32)
    hlv_pnT = jnp.dot(hlvsT_ref[...], oh, preferred_element_type=F32)
    n_node = stT_ref.shape[0]
    n_hlvs = hlvsT_ref.shape[0]
    n_all = outT_ref.shape[0]
    outT_ref[...] = (
        jnp.dot(_row_proj(n_all, 1, 0), xT_ref[...],
                preferred_element_type=F32)
        + jnp.dot(_row_proj(n_all, n_node, 4), stT_ref[...],
                  preferred_element_type=F32)
        + jnp.dot(_row_proj(n_all, n_hlvs, 4 + n_node), hlv_pnT,
                  preferred_element_type=F32))


def _round_coreT(aT_ref, pT_ref, eps_ref, wrefs):
    (cw1, cw2, cw3, cw4, cb1, cb2, cb3, cb4,
     nw1d, nw1s, nw2, nw3, nw4, nb1, nb2, nb3, nb4) = wrefs
    aT = aT_ref[...]
    m = (1.0 + eps_ref[0, 0]) * aT + pT_ref[...]
    h4 = _mlpT(m, (cw1, cw2, cw3, cw4), (cb1, cb2, cb3, cb4))
    n_dyn = h4.shape[0]
    a_staticT = aT[n_dyn:, :]
    z = jnp.maximum(
        jnp.dot(nw1d[...], h4, preferred_element_type=F32)
        + jnp.dot(nw1s[...], a_staticT, preferred_element_type=F32)
        + nb1[...], 0.0)
    h2 = _mlpT(z, (nw2, nw3, nw4), (nb2, nb3, nb4))
    return h2, a_staticT


def _round1_body(aT_ref, pT_ref, eps_ref, *wrefs_and_out):
    *wrefs, outT_ref = wrefs_and_out
    h2, a_staticT = _round_coreT(aT_ref, pT_ref, eps_ref, wrefs)
    n_all = aT_ref.shape[0]
    n_dyn = h2.shape[0]
    outT_ref[...] = (
        jnp.dot(_row_proj(n_all, n_dyn, 0), h2, preferred_element_type=F32)
        + jnp.dot(_row_proj(n_all, n_all - n_dyn, n_dyn), a_staticT,
                  preferred_element_type=F32))


def _round2_body(aT_ref, pT_ref, eps_ref, bT_ref, hlvsT_ref, *rest,
                 nblocks, g):
    (cw1, cw2, cw3, cw4, cb1, cb2, cb3, cb4,
     nw1d, nw1s, nw2, nw3, nw4, nb1, nb2, nb3, nb4,
     hw1, hw2, hw3, hw4, hb1, hb2, hb3, hb4,
     pooledT_ref, outT_ref) = rest
    wrefs = (cw1, cw2, cw3, cw4, cb1, cb2, cb3, cb4,
             nw1d, nw1s, nw2, nw3, nw4, nb1, nb2, nb3, nb4)
    h2, _ = _round_coreT(aT_ref, pT_ref, eps_ref, wrefs)
    b = h2.shape[1]
    i = pl.program_id(0)

    oh = (lax.broadcasted_iota(jnp.int32, (g, b), 0)
          == bT_ref[...]).astype(F32)
    partial = lax.dot_general(
        h2, oh, (((1,), (1,)), ((), ())), preferred_element_type=F32)

    @pl.when(i == 0)
    def _():
        pooledT_ref[...] = jnp.zeros_like(pooledT_ref)

    pooledT_ref[...] += partial

    @pl.when(i == nblocks - 1)
    def _():
        pooledT = pooledT_ref[...]
        n_hlvs = hlvsT_ref.shape[0]
        n_dyn = pooledT.shape[0]
        tot = n_hlvs + n_dyn
        zT = (jnp.dot(_row_proj(tot, n_hlvs, 0), hlvsT_ref[...],
                      preferred_element_type=F32)
              + jnp.dot(_row_proj(tot, n_dyn, n_hlvs), pooledT,
                        preferred_element_type=F32))
        outT_ref[...] = _mlpT(zT, (hw1, hw2, hw3, hw4),
                              (hb1, hb2, hb3, hb4))


def kernel(x, feature_mtx_static, hlvs, batch, edge_index, eps,
           conv_W, conv_b, node_W, node_b, hlv_W, hlv_b):
    n = x.shape[0]
    e = edge_index.shape[1]
    g = hlvs.shape[0]
    n_node = feature_mtx_static.shape[1]
    n_hlvs = hlvs.shape[1]
    n_dyn = conv_W[-1].shape[1]
    n_all = n_dyn + n_node + n_hlvs

    nblocks = -(-n // BLK)
    n_pad = nblocks * BLK
    half = n_pad // 2
    assert half % NUM_SUBCORES == 0
    assert (half + LANES) % NUM_SUBCORES == 0

    n_rows = -(-e // EDGE_LANES)
    rows_per_tile = _ceil_to(-(-n_rows // NUM_SUBCORES), K_GROUP)
    r_pad = rows_per_tile * NUM_SUBCORES
    pad_e = r_pad * EDGE_LANES - e
    src2d = jnp.concatenate(
        [edge_index[0], jnp.zeros((pad_e,), jnp.int32)]).reshape(
            r_pad, EDGE_LANES)
    dst2d = jnp.concatenate(
        [edge_index[1], jnp.full((pad_e,), n, jnp.int32)]).reshape(
            r_pad, EDGE_LANES)

    segsum = _make_edge_segsum(half, rows_per_tile, n_all)

    xT = jnp.pad(jnp.reshape(x, (1, n)), ((0, 0), (0, n_pad - n)))
    stT = jnp.pad(jnp.transpose(feature_mtx_static),
                  ((0, 0), (0, n_pad - n)))
    bT = jnp.pad(jnp.reshape(batch, (1, n)), ((0, 0), (0, n_pad - n)),
                 constant_values=g)
    hlvsT = jnp.transpose(hlvs)

    w_ops = []
    cwT = [jnp.transpose(w) for w in conv_W]
    nwT = [jnp.transpose(w) for w in node_W]
    w_ops += cwT
    w_ops += [jnp.reshape(b, (-1, 1)) for b in conv_b]
    w_ops += [nwT[0][:, :n_dyn], nwT[0][:, n_dyn:], nwT[1], nwT[2], nwT[3]]
    w_ops += [jnp.reshape(b, (-1, 1)) for b in node_b]
    w_in = [pl.BlockSpec(a.shape, lambda i: (0, 0)) for a in w_ops]
    hw_ops = [jnp.transpose(w) for w in hlv_W]
    hw_ops += [jnp.reshape(b, (-1, 1)) for b in hlv_b]
    hw_in = [pl.BlockSpec(a.shape, lambda i: (0, 0)) for a in hw_ops]
    eps2d = jnp.reshape(eps, (1, 1))
    eps_in = pl.BlockSpec((1, 1), lambda i: (0, 0))

    row_spec = lambda rows: pl.BlockSpec((rows, BLK), lambda i: (0, i))
    aT_spec = row_spec(n_all)

    a0T = pl.pallas_call(
        functools.partial(_prep_body, g=g),
        grid=(nblocks,),
        in_specs=[row_spec(1), row_spec(n_node), row_spec(1),
                  pl.BlockSpec((n_hlvs, g), lambda i: (0, 0))],
        out_specs=aT_spec,
        out_shape=jax.ShapeDtypeStruct((n_all, n_pad), F32),
    )(xT, stT, bT, hlvsT)

    p0 = segsum(jnp.transpose(a0T), src2d, dst2d).reshape(n_pad, n_all)
    a1T = pl.pallas_call(
        _round1_body,
        grid=(nblocks,),
        in_specs=[aT_spec, aT_spec, eps_in] + w_in,
        out_specs=aT_spec,
        out_shape=jax.ShapeDtypeStruct((n_all, n_pad), F32),
    )(a0T, jnp.transpose(p0), eps2d, *w_ops)

    p1 = segsum(jnp.transpose(a1T), src2d, dst2d).reshape(n_pad, n_all)
    _, outT = pl.pallas_call(
        functools.partial(_round2_body, nblocks=nblocks, g=g),
        grid=(nblocks,),
        in_specs=[aT_spec, aT_spec, eps_in, row_spec(1),
                  pl.BlockSpec((n_hlvs, g), lambda i: (0, 0))]
        + w_in + hw_in,
        out_specs=[pl.BlockSpec((n_dyn, g), lambda i: (0, 0)),
                   pl.BlockSpec((1, g), lambda i: (0, 0))],
        out_shape=[jax.ShapeDtypeStruct((n_dyn, g), F32),
                   jax.ShapeDtypeStruct((1, g), F32)],
    )(a1T, jnp.transpose(p1), eps2d, bT, hlvsT, *w_ops, *hw_ops)
    return jnp.reshape(outT, (g, 1))


# speedup vs baseline: 28.3417x; 1.2391x over previous
import functools

import jax
import jax.numpy as jnp
from jax import lax
from jax.experimental import pallas as pl
from jax.experimental.pallas import tpu as pltpu
from jax.experimental.pallas import tpu_sc as plsc

NUM_CORES = 2
NUM_SUBCORES = 16
LANES = 16
EDGE_LANES = 128
K_GROUP = 16
ZROWS = 256

BLK = 2048

F32 = jnp.float32


def _ceil_to(x, m):
    return (x + m - 1) // m * m


def _make_edge_segsum(half, rows_per_tile, feat):
    mesh = plsc.VectorSubcoreMesh(core_axis_name="c", subcore_axis_name="s")
    pairs = rows_per_tile // (2 * K_GROUP)
    n_acc = half + LANES
    zpt = n_acc // NUM_SUBCORES
    wpt = half // NUM_SUBCORES

    @functools.partial(
        pl.kernel,
        out_type=jax.ShapeDtypeStruct((NUM_CORES, half, feat), F32),
        mesh=mesh,
        scratch_types=[
            pltpu.VMEM((K_GROUP, EDGE_LANES), jnp.int32),
            pltpu.VMEM((K_GROUP, EDGE_LANES), jnp.int32),
            pltpu.VMEM((K_GROUP, EDGE_LANES), jnp.int32),
            pltpu.VMEM((K_GROUP, EDGE_LANES), jnp.int32),
            pltpu.VMEM((K_GROUP, EDGE_LANES, feat), F32),
            pltpu.VMEM((K_GROUP, EDGE_LANES, feat), F32),
            pltpu.VMEM((ZROWS, feat), F32),
            pltpu.VMEM_SHARED((n_acc, feat), F32),
            pltpu.SemaphoreType.DMA,
            pltpu.SemaphoreType.DMA,
            pltpu.SemaphoreType.DMA,
            pltpu.SemaphoreType.DMA,
        ],
        compiler_params=pltpu.CompilerParams(use_tc_tiling_on_sc=False),
    )
    def seg(a_hbm, src_hbm, dst_hbm, out_hbm, src_a, dst_a, src_b, dst_b,
            rows_a, rows_b, zbuf_v, acc_sh, sem_ga, sem_gb, sem_sa, sem_sb):
        c = lax.axis_index("c")
        s = lax.axis_index("s")
        lo = c * half

        def zfill(i, _):
            zbuf_v[i] = jnp.zeros((feat,), F32)
            return 0

        lax.fori_loop(0, ZROWS, zfill, 0)
        zrow0 = s * zpt
        nz, rem = zpt // ZROWS, zpt % ZROWS
        zcopies = [
            pltpu.async_copy(
                zbuf_v, acc_sh.at[pl.ds(zrow0 + k * ZROWS, ZROWS)], sem_sa)
            for k in range(nz)
        ]
        if rem:
            zcopies.append(
                pltpu.async_copy(
                    zbuf_v.at[pl.ds(0, rem)],
                    acc_sh.at[pl.ds(zrow0 + nz * ZROWS, rem)], sem_sa))
        for d in zcopies:
            d.wait()
        plsc.subcore_barrier()

        row0 = s * rows_per_tile

        def clamp(dst_v):
            for j in range(K_GROUP):
                for k in range(EDGE_LANES // LANES):
                    d = dst_v[j, pl.ds(k * LANES, LANES)]
                    dl = d - lo
                    inb = jnp.less(
                        lax.bitcast_convert_type(dl, jnp.uint32),
                        jnp.uint32(half))
                    dst_v[j, pl.ds(k * LANES, LANES)] = jnp.where(
                        inb, dl, half + (d & (LANES - 1)))

        def grp(gi, _):
            base_a = row0 + gi * 2 * K_GROUP
            base_b = base_a + K_GROUP
            pltpu.sync_copy(src_hbm.at[pl.ds(base_a, K_GROUP)], src_a)
            pltpu.sync_copy(dst_hbm.at[pl.ds(base_a, K_GROUP)], dst_a)
            ga = [
                pltpu.async_copy(a_hbm.at[src_a.at[j]], rows_a.at[j], sem_ga)
                for j in range(K_GROUP)
            ]
            clamp(dst_a)
            pltpu.sync_copy(src_hbm.at[pl.ds(base_b, K_GROUP)], src_b)
            pltpu.sync_copy(dst_hbm.at[pl.ds(base_b, K_GROUP)], dst_b)
            gb = [
                pltpu.async_copy(a_hbm.at[src_b.at[j]], rows_b.at[j], sem_gb)
                for j in range(K_GROUP)
            ]
            clamp(dst_b)
            for d in ga:
                d.wait()
            sa = [
                pltpu.async_copy(
                    rows_a.at[j], acc_sh.at[dst_a.at[j]], sem_sa, add=True)
                for j in range(K_GROUP)
            ]
            for d in gb:
                d.wait()
            sb = [
                pltpu.async_copy(
                    rows_b.at[j], acc_sh.at[dst_b.at[j]], sem_sb, add=True)
                for j in range(K_GROUP)
            ]
            for d in sa:
                d.wait()
            for d in sb:
                d.wait()
            return 0

        lax.fori_loop(0, pairs, grp, 0)
        plsc.subcore_barrier()

        pltpu.sync_copy(
            acc_sh.at[pl.ds(s * wpt, wpt)],
            out_hbm.at[c, pl.ds(s * wpt, wpt)])

    return seg


def _row_proj(total, rows_from, offset):
    r = lax.broadcasted_iota(jnp.int32, (total, rows_from), 0)
    c = lax.broadcasted_iota(jnp.int32, (total, rows_from), 1)
    return (r - offset == c).astype(F32)


def _mlpT(h, w_refs, b_refs):
    for w_ref, b_ref in zip(w_refs, b_refs):
        h = jnp.maximum(jnp.dot(w_ref[...], h,
                                preferred_element_type=F32) + b_ref[...], 0.0)
    return h


def _prep_body(xT_ref, stT_ref, bT_ref, hlvsT_ref, outT_ref, *, g):
    b = xT_ref.shape[1]
    oh = (lax.broadcasted_iota(jnp.int32, (g, b), 0)
          == bT_ref[...]).astype(F32)
    hlv_pnT = jnp.dot(hlvsT_ref[...], oh, preferred_element_type=F32)
    n_node = stT_ref.shape[0]
    n_hlvs = hlvsT_ref.shape[0]
    n_all = outT_ref.shape[0]
    outT_ref[...] = (
        jnp.dot(_row_proj(n_all, 1, 0), xT_ref[...],
                preferred_element_type=F32)
        + jnp.dot(_row_proj(n_all, n_node, 4), stT_ref[...],
                  preferred_element_type=F32)
        + jnp.dot(_row_proj(n_all, n_hlvs, 4 + n_node), hlv_pnT,
                  preferred_element_type=F32))


def _round_coreT(aT_ref, pT_ref, eps_ref, wrefs):
    (cw1, cw2, cw3, cw4, cb1, cb2, cb3, cb4,
     nw1d, nw1s, nw2, nw3, nw4, nb1, nb2, nb3, nb4) = wrefs
    aT = aT_ref[...]
    m = (1.0 + eps_ref[0, 0]) * aT + pT_ref[...]
    h4 = _mlpT(m, (cw1, cw2, cw3, cw4), (cb1, cb2, cb3, cb4))
    n_dyn = h4.shape[0]
    a_staticT = aT[n_dyn:, :]
    z = jnp.maximum(
        jnp.dot(nw1d[...], h4, preferred_element_type=F32)
        + jnp.dot(nw1s[...], a_staticT, preferred_element_type=F32)
        + nb1[...], 0.0)
    h2 = _mlpT(z, (nw2, nw3, nw4), (nb2, nb3, nb4))
    return h2, a_staticT


def _round1_body(aT_ref, pT_ref, eps_ref, *wrefs_and_out):
    *wrefs, outT_ref = wrefs_and_out
    h2, a_staticT = _round_coreT(aT_ref, pT_ref, eps_ref, wrefs)
    n_all = aT_ref.shape[0]
    n_dyn = h2.shape[0]
    outT_ref[...] = (
        jnp.dot(_row_proj(n_all, n_dyn, 0), h2, preferred_element_type=F32)
        + jnp.dot(_row_proj(n_all, n_all - n_dyn, n_dyn), a_staticT,
                  preferred_element_type=F32))


def _round2_body(aT_ref, pT_ref, eps_ref, bT_ref, hlvsT_ref, *rest,
                 nblocks, g):
    (cw1, cw2, cw3, cw4, cb1, cb2, cb3, cb4,
     nw1d, nw1s, nw2, nw3, nw4, nb1, nb2, nb3, nb4,
     hw1, hw2, hw3, hw4, hb1, hb2, hb3, hb4,
     pooledT_ref, outT_ref) = rest
    wrefs = (cw1, cw2, cw3, cw4, cb1, cb2, cb3, cb4,
             nw1d, nw1s, nw2, nw3, nw4, nb1, nb2, nb3, nb4)
    h2, _ = _round_coreT(aT_ref, pT_ref, eps_ref, wrefs)
    b = h2.shape[1]
    i = pl.program_id(0)

    oh = (lax.broadcasted_iota(jnp.int32, (g, b), 0)
          == bT_ref[...]).astype(F32)
    partial = lax.dot_general(
        h2, oh, (((1,), (1,)), ((), ())), preferred_element_type=F32)

    @pl.when(i == 0)
    def _():
        pooledT_ref[...] = jnp.zeros_like(pooledT_ref)

    pooledT_ref[...] += partial

    @pl.when(i == nblocks - 1)
    def _():
        pooledT = pooledT_ref[...]
        n_hlvs = hlvsT_ref.shape[0]
        n_dyn = pooledT.shape[0]
        tot = n_hlvs + n_dyn
        zT = (jnp.dot(_row_proj(tot, n_hlvs, 0), hlvsT_ref[...],
                      preferred_element_type=F32)
              + jnp.dot(_row_proj(tot, n_dyn, n_hlvs), pooledT,
                        preferred_element_type=F32))
        outT_ref[...] = _mlpT(zT, (hw1, hw2, hw3, hw4),
                              (hb1, hb2, hb3, hb4))


def kernel(x, feature_mtx_static, hlvs, batch, edge_index, eps,
           conv_W, conv_b, node_W, node_b, hlv_W, hlv_b):
    n = x.shape[0]
    e = edge_index.shape[1]
    g = hlvs.shape[0]
    n_node = feature_mtx_static.shape[1]
    n_hlvs = hlvs.shape[1]
    n_dyn = conv_W[-1].shape[1]
    n_all = n_dyn + n_node + n_hlvs

    nblocks = -(-n // BLK)
    n_pad = nblocks * BLK
    half = n_pad // 2
    assert half % NUM_SUBCORES == 0
    assert (half + LANES) % NUM_SUBCORES == 0

    n_rows = -(-e // EDGE_LANES)
    rows_per_tile = _ceil_to(-(-n_rows // NUM_SUBCORES), 2 * K_GROUP)
    r_pad = rows_per_tile * NUM_SUBCORES
    pad_e = r_pad * EDGE_LANES - e
    src2d = jnp.concatenate(
        [edge_index[0], jnp.zeros((pad_e,), jnp.int32)]).reshape(
            r_pad, EDGE_LANES)
    dst2d = jnp.concatenate(
        [edge_index[1], jnp.full((pad_e,), n, jnp.int32)]).reshape(
            r_pad, EDGE_LANES)

    segsum = _make_edge_segsum(half, rows_per_tile, n_all)

    xT = jnp.pad(jnp.reshape(x, (1, n)), ((0, 0), (0, n_pad - n)))
    stT = jnp.pad(jnp.transpose(feature_mtx_static),
                  ((0, 0), (0, n_pad - n)))
    bT = jnp.pad(jnp.reshape(batch, (1, n)), ((0, 0), (0, n_pad - n)),
                 constant_values=g)
    hlvsT = jnp.transpose(hlvs)

    w_ops = []
    cwT = [jnp.transpose(w) for w in conv_W]
    nwT = [jnp.transpose(w) for w in node_W]
    w_ops += cwT
    w_ops += [jnp.reshape(b, (-1, 1)) for b in conv_b]
    w_ops += [nwT[0][:, :n_dyn], nwT[0][:, n_dyn:], nwT[1], nwT[2], nwT[3]]
    w_ops += [jnp.reshape(b, (-1, 1)) for b in node_b]
    w_in = [pl.BlockSpec(a.shape, lambda i: (0, 0)) for a in w_ops]
    hw_ops = [jnp.transpose(w) for w in hlv_W]
    hw_ops += [jnp.reshape(b, (-1, 1)) for b in hlv_b]
    hw_in = [pl.BlockSpec(a.shape, lambda i: (0, 0)) for a in hw_ops]
    eps2d = jnp.reshape(eps, (1, 1))
    eps_in = pl.BlockSpec((1, 1), lambda i: (0, 0))

    row_spec = lambda rows: pl.BlockSpec((rows, BLK), lambda i: (0, i))
    aT_spec = row_spec(n_all)

    a0T = pl.pallas_call(
        functools.partial(_prep_body, g=g),
        grid=(nblocks,),
        in_specs=[row_spec(1), row_spec(n_node), row_spec(1),
                  pl.BlockSpec((n_hlvs, g), lambda i: (0, 0))],
        out_specs=aT_spec,
        out_shape=jax.ShapeDtypeStruct((n_all, n_pad), F32),
    )(xT, stT, bT, hlvsT)

    p0 = segsum(jnp.transpose(a0T), src2d, dst2d).reshape(n_pad, n_all)
    a1T = pl.pallas_call(
        _round1_body,
        grid=(nblocks,),
        in_specs=[aT_spec, aT_spec, eps_in] + w_in,
        out_specs=aT_spec,
        out_shape=jax.ShapeDtypeStruct((n_all, n_pad), F32),
    )(a0T, jnp.transpose(p0), eps2d, *w_ops)

    p1 = segsum(jnp.transpose(a1T), src2d, dst2d).reshape(n_pad, n_all)
    _, outT = pl.pallas_call(
        functools.partial(_round2_body, nblocks=nblocks, g=g),
        grid=(nblocks,),
        in_specs=[aT_spec, aT_spec, eps_in, row_spec(1),
                  pl.BlockSpec((n_hlvs, g), lambda i: (0, 0))]
        + w_in + hw_in,
        out_specs=[pl.BlockSpec((n_dyn, g), lambda i: (0, 0)),
                   pl.BlockSpec((1, g), lambda i: (0, 0))],
        out_shape=[jax.ShapeDtypeStruct((n_dyn, g), F32),
                   jax.ShapeDtypeStruct((1, g), F32)],
    )(a1T, jnp.transpose(p1), eps2d, bT, hlvsT, *w_ops, *hw_ops)
    return jnp.reshape(outT, (g, 1))
